# baseline (speedup 1.0000x reference)
import functools

import jax
import jax.numpy as jnp
from jax import lax
from jax.experimental import pallas as pl
from jax.experimental.pallas import tpu as pltpu
from jax.experimental.pallas import tpu_sc as plsc

HID = 768
LANES = 16
VPR = HID // LANES
NW = 32
CHUNK = 32
EPS = 1e-5


def _allsum16(x):
    idx = lax.iota(jnp.int32, LANES)
    dnums = lax.GatherDimensionNumbers(
        offset_dims=(), collapsed_slice_dims=(0,), start_index_map=(0,))
    for k in (8, 4, 2, 1):
        x = x + lax.gather(x, (idx ^ k)[:, None], dnums, slice_sizes=(1,),
                           mode=lax.GatherScatterMode.PROMISE_IN_BOUNDS)
    return x


def _rsqrt16(x):
    i = lax.bitcast_convert_type(x, jnp.int32)
    y = lax.bitcast_convert_type(jnp.int32(0x5F3759DF) - (i >> 1), jnp.float32)
    for _ in range(3):
        y = y * (1.5 - 0.5 * x * y * y)
    return y


def _emb_body(tok_t, seg_t, pos_t, tid, sid, pid, gam, bet, out,
              idx_t, idx_s, idx_p, rt, rs, rp, gv, bv, sem, tpw, nchunk):
    wid = lax.axis_index("s") * 2 + lax.axis_index("c")
    base = wid * tpw
    pltpu.sync_copy(tid.at[pl.ds(base, tpw)], idx_t)
    pltpu.sync_copy(sid.at[pl.ds(base, tpw)], idx_s)
    pltpu.sync_copy(pid.at[pl.ds(base, tpw)], idx_p)
    pltpu.sync_copy(gam, gv)
    pltpu.sync_copy(bet, bv)

    def chunk(c, carry):
        off = c * CHUNK
        cp_t = pltpu.async_copy(tok_t.at[idx_t.at[pl.ds(off, CHUNK)]], rt, sem)
        cp_s = pltpu.async_copy(seg_t.at[idx_s.at[pl.ds(off, CHUNK)]], rs, sem)
        cp_p = pltpu.async_copy(pos_t.at[idx_p.at[pl.ds(off, CHUNK)]], rp, sem)
        cp_t.wait()
        cp_s.wait()
        cp_p.wait()

        def token(i, tc):
            s = jnp.zeros((LANES,), jnp.float32)
            ss = jnp.zeros((LANES,), jnp.float32)
            for j in range(VPR):
                sl = pl.ds(j * LANES, LANES)
                x = rt[i, sl] + rs[i, sl] + rp[i, sl]
                rt[i, sl] = x
                s = s + x
                ss = ss + x * x
            mean = _allsum16(s) * (1.0 / HID)
            msq = _allsum16(ss) * (1.0 / HID)
            inv = _rsqrt16(msq - mean * mean + EPS)
            for j in range(VPR):
                sl = pl.ds(j * LANES, LANES)
                rt[i, sl] = (rt[i, sl] - mean) * (inv * gv[sl]) + bv[sl]
            return tc

        lax.fori_loop(0, CHUNK, token, 0)
        pltpu.sync_copy(rt, out.at[pl.ds(base + off, CHUNK)])
        return carry

    lax.fori_loop(0, nchunk, chunk, 0)


def kernel(token_ids, segment_ids, position_ids, tok_table, seg_table,
           pos_table, gamma, beta):
    b, s = token_ids.shape
    n = b * s
    tpw = n // NW
    nchunk = tpw // CHUNK
    tid = token_ids.reshape(n).astype(jnp.int32)
    sid = segment_ids.reshape(n).astype(jnp.int32)
    pid = position_ids.reshape(n).astype(jnp.int32)

    body = functools.partial(_emb_body, tpw=tpw, nchunk=nchunk)
    fn = pl.kernel(
        body,
        mesh=plsc.VectorSubcoreMesh(core_axis_name="c", subcore_axis_name="s"),
        out_type=jax.ShapeDtypeStruct((n, HID), jnp.float32),
        scratch_types=[
            pltpu.VMEM((tpw,), jnp.int32),
            pltpu.VMEM((tpw,), jnp.int32),
            pltpu.VMEM((tpw,), jnp.int32),
            pltpu.VMEM((CHUNK, HID), jnp.float32),
            pltpu.VMEM((CHUNK, HID), jnp.float32),
            pltpu.VMEM((CHUNK, HID), jnp.float32),
            pltpu.VMEM((HID,), jnp.float32),
            pltpu.VMEM((HID,), jnp.float32),
            pltpu.SemaphoreType.DMA,
        ],
    )
    out = fn(tok_table, seg_table, pos_table, tid, sid, pid, gamma, beta)
    return out.reshape(b, s, HID)


# speedup vs baseline: 1.3590x; 1.3590x over previous
import functools

import jax
import jax.numpy as jnp
from jax import lax
from jax.experimental import pallas as pl
from jax.experimental.pallas import tpu as pltpu
from jax.experimental.pallas import tpu_sc as plsc

HID = 768
LANES = 16
VPR = HID // LANES
NW = 32
CHUNK = 32
EPS = 1e-5


def _allsum16(x):
    idx = lax.iota(jnp.int32, LANES)
    dnums = lax.GatherDimensionNumbers(
        offset_dims=(), collapsed_slice_dims=(0,), start_index_map=(0,))
    for k in (8, 4, 2, 1):
        x = x + lax.gather(x, (idx ^ k)[:, None], dnums, slice_sizes=(1,),
                           mode=lax.GatherScatterMode.PROMISE_IN_BOUNDS)
    return x


def _rsqrt16(x):
    i = lax.bitcast_convert_type(x, jnp.int32)
    y = lax.bitcast_convert_type(jnp.int32(0x5F3759DF) - (i >> 1), jnp.float32)
    for _ in range(3):
        y = y * (1.5 - 0.5 * x * y * y)
    return y


def _emb_body(tok_t, seg_t, pos_t, tid, sid, pid, gam, bet, out,
              idx_t, idx_s, idx_p, rt, rp, sv, gv, bv, sem, tpw, nchunk):
    wid = lax.axis_index("s") * 2 + lax.axis_index("c")
    base = wid * tpw
    pltpu.sync_copy(tid.at[pl.ds(base, tpw)], idx_t)
    pltpu.sync_copy(sid.at[pl.ds(base, tpw)], idx_s.at[pl.ds(0, tpw)])
    pltpu.sync_copy(pid.at[pl.ds(base, tpw)], idx_p)
    pltpu.sync_copy(gam, gv)
    pltpu.sync_copy(bet, bv)

    pltpu.sync_copy(seg_t, sv)

    def chunk(c, carry):
        off = c * CHUNK
        cp_t = pltpu.async_copy(tok_t.at[idx_t.at[pl.ds(off, CHUNK)]], rt, sem)
        cp_p = pltpu.async_copy(pos_t.at[idx_p.at[pl.ds(off, CHUNK)]], rp, sem)
        cp_t.wait()
        cp_p.wait()

        def token(i, tc):
            sid = idx_s[pl.ds(off + i, LANES)][0]
            s = jnp.zeros((LANES,), jnp.float32)
            ss = jnp.zeros((LANES,), jnp.float32)
            for j in range(VPR):
                sl = pl.ds(j * LANES, LANES)
                x = rt[i, sl] + rp[i, sl] + sv[sid, sl]
                rt[i, sl] = x
                s = s + x
                ss = ss + x * x
            mean = _allsum16(s) * (1.0 / HID)
            msq = _allsum16(ss) * (1.0 / HID)
            inv = _rsqrt16(msq - mean * mean + EPS)
            for j in range(VPR):
                sl = pl.ds(j * LANES, LANES)
                rt[i, sl] = (rt[i, sl] - mean) * (inv * gv[sl]) + bv[sl]
            return tc

        lax.fori_loop(0, CHUNK, token, 0)
        pltpu.sync_copy(rt, out.at[pl.ds(base + off, CHUNK)])
        return carry

    lax.fori_loop(0, nchunk, chunk, 0)


def kernel(token_ids, segment_ids, position_ids, tok_table, seg_table,
           pos_table, gamma, beta):
    b, s = token_ids.shape
    n = b * s
    tpw = n // NW
    nchunk = tpw // CHUNK
    tid = token_ids.reshape(n).astype(jnp.int32)
    sid = segment_ids.reshape(n).astype(jnp.int32)
    pid = position_ids.reshape(n).astype(jnp.int32)

    body = functools.partial(_emb_body, tpw=tpw, nchunk=nchunk)
    fn = pl.kernel(
        body,
        mesh=plsc.VectorSubcoreMesh(core_axis_name="c", subcore_axis_name="s"),
        out_type=jax.ShapeDtypeStruct((n, HID), jnp.float32),
        scratch_types=[
            pltpu.VMEM((tpw,), jnp.int32),
            pltpu.VMEM((tpw + LANES,), jnp.int32),
            pltpu.VMEM((tpw,), jnp.int32),
            pltpu.VMEM((CHUNK, HID), jnp.float32),
            pltpu.VMEM((CHUNK, HID), jnp.float32),
            pltpu.VMEM((2, HID), jnp.float32),
            pltpu.VMEM((HID,), jnp.float32),
            pltpu.VMEM((HID,), jnp.float32),
            pltpu.SemaphoreType.DMA,
        ],
    )
    out = fn(tok_table, seg_table, pos_table, tid, sid, pid, gamma, beta)
    return out.reshape(b, s, HID)


# speedup vs baseline: 1.5266x; 1.1233x over previous
import functools

import jax
import jax.numpy as jnp
from jax import lax
from jax.experimental import pallas as pl
from jax.experimental.pallas import tpu as pltpu
from jax.experimental.pallas import tpu_sc as plsc

HID = 768
LANES = 16
VPR = HID // LANES
NW = 32
CHUNK = 16
EPS = 1e-5


def _allsum16(x):
    idx = lax.iota(jnp.int32, LANES)
    dnums = lax.GatherDimensionNumbers(
        offset_dims=(), collapsed_slice_dims=(0,), start_index_map=(0,))
    for k in (8, 4, 2, 1):
        x = x + lax.gather(x, (idx ^ k)[:, None], dnums, slice_sizes=(1,),
                           mode=lax.GatherScatterMode.PROMISE_IN_BOUNDS)
    return x


def _rsqrt16(x):
    i = lax.bitcast_convert_type(x, jnp.int32)
    y = lax.bitcast_convert_type(jnp.int32(0x5F3759DF) - (i >> 1), jnp.float32)
    for _ in range(3):
        y = y * (1.5 - 0.5 * x * y * y)
    return y


def _emb_body(tok_t, seg_t, pos_t, tid, sid, pid, gam, bet, out,
              idx_t, idx_s, idx_p, rt0, rp0, rt1, rp1, ro0, ro1, sv, gv, bv,
              sg0, sg1, ss0, ss1, tpw, nchunk):
    wid = lax.axis_index("s") * 2 + lax.axis_index("c")
    base = wid * tpw
    pltpu.sync_copy(tid.at[pl.ds(base, tpw)], idx_t)
    pltpu.sync_copy(sid.at[pl.ds(base, tpw)], idx_s.at[pl.ds(0, tpw)])
    pltpu.sync_copy(pid.at[pl.ds(base, tpw)], idx_p)
    pltpu.sync_copy(gam, gv)
    pltpu.sync_copy(bet, bv)
    pltpu.sync_copy(seg_t, sv)

    def gstart(c, rt, rp, sem):
        off = c * CHUNK
        pltpu.async_copy(tok_t.at[idx_t.at[pl.ds(off, CHUNK)]], rt, sem)
        pltpu.async_copy(pos_t.at[idx_p.at[pl.ds(off, CHUNK)]], rp, sem)

    def gwait(rt, rp, sem):
        pltpu.make_async_copy(tok_t.at[pl.ds(0, CHUNK)], rt, sem).wait()
        pltpu.make_async_copy(pos_t.at[pl.ds(0, CHUNK)], rp, sem).wait()

    def swait(ro, sem):
        pltpu.make_async_copy(ro, out.at[pl.ds(0, CHUNK)], sem).wait()

    def compute(rt, rp, ro, off):
        def token(i, tc):
            sid_ = idx_s[pl.ds(off + i, LANES)][0]
            s = jnp.zeros((LANES,), jnp.float32)
            ssq = jnp.zeros((LANES,), jnp.float32)
            for j in range(VPR):
                sl = pl.ds(j * LANES, LANES)
                x = rt[i, sl] + rp[i, sl] + sv[sid_, sl]
                ro[i, sl] = x
                s = s + x
                ssq = ssq + x * x
            mean = _allsum16(s) * (1.0 / HID)
            msq = _allsum16(ssq) * (1.0 / HID)
            inv = _rsqrt16(msq - mean * mean + EPS)
            for j in range(VPR):
                sl = pl.ds(j * LANES, LANES)
                ro[i, sl] = (ro[i, sl] - mean) * (inv * gv[sl]) + bv[sl]
            return tc

        lax.fori_loop(0, CHUNK, token, 0)

    gstart(0, rt0, rp0, sg0)
    gstart(1, rt1, rp1, sg1)
    nc2 = nchunk // 2

    def pair(c2, carry):
        e = c2 * 2
        o = e + 1
        gwait(rt0, rp0, sg0)

        @pl.when(c2 > 0)
        def _():
            swait(ro0, ss0)

        compute(rt0, rp0, ro0, e * CHUNK)
        pltpu.async_copy(ro0, out.at[pl.ds(base + e * CHUNK, CHUNK)], ss0)

        @pl.when(c2 + 1 < nc2)
        def _():
            gstart(e + 2, rt0, rp0, sg0)

        gwait(rt1, rp1, sg1)

        @pl.when(c2 > 0)
        def _():
            swait(ro1, ss1)

        compute(rt1, rp1, ro1, o * CHUNK)
        pltpu.async_copy(ro1, out.at[pl.ds(base + o * CHUNK, CHUNK)], ss1)

        @pl.when(c2 + 1 < nc2)
        def _():
            gstart(o + 2, rt1, rp1, sg1)

        return carry

    lax.fori_loop(0, nc2, pair, 0)
    swait(ro0, ss0)
    swait(ro1, ss1)


def kernel(token_ids, segment_ids, position_ids, tok_table, seg_table,
           pos_table, gamma, beta):
    b, s = token_ids.shape
    n = b * s
    tpw = n // NW
    nchunk = tpw // CHUNK
    tid = token_ids.reshape(n).astype(jnp.int32)
    sid = segment_ids.reshape(n).astype(jnp.int32)
    pid = position_ids.reshape(n).astype(jnp.int32)

    body = functools.partial(_emb_body, tpw=tpw, nchunk=nchunk)
    fn = pl.kernel(
        body,
        mesh=plsc.VectorSubcoreMesh(core_axis_name="c", subcore_axis_name="s"),
        out_type=jax.ShapeDtypeStruct((n, HID), jnp.float32),
        scratch_types=[
            pltpu.VMEM((tpw,), jnp.int32),
            pltpu.VMEM((tpw + LANES,), jnp.int32),
            pltpu.VMEM((tpw,), jnp.int32),
            pltpu.VMEM((CHUNK, HID), jnp.float32),
            pltpu.VMEM((CHUNK, HID), jnp.float32),
            pltpu.VMEM((CHUNK, HID), jnp.float32),
            pltpu.VMEM((CHUNK, HID), jnp.float32),
            pltpu.VMEM((CHUNK, HID), jnp.float32),
            pltpu.VMEM((CHUNK, HID), jnp.float32),
            pltpu.VMEM((2, HID), jnp.float32),
            pltpu.VMEM((HID,), jnp.float32),
            pltpu.VMEM((HID,), jnp.float32),
            pltpu.SemaphoreType.DMA,
            pltpu.SemaphoreType.DMA,
            pltpu.SemaphoreType.DMA,
            pltpu.SemaphoreType.DMA,
        ],
    )
    out = fn(tok_table, seg_table, pos_table, tid, sid, pid, gamma, beta)
    return out.reshape(b, s, HID)
